# baseline (speedup 1.0000x reference)
import functools

import jax
import jax.numpy as jnp
from jax import lax
from jax.experimental import pallas as pl
from jax.experimental.pallas import tpu as pltpu
from jax.experimental.pallas import tpu_sc as plsc

NC = 2
NS = 16
NW = NC * NS
CHUNK = 128
BLK = 1024
SHIFT = 15


def _round_up(a, b):
    return (a + b - 1) // b * b


def _sc_degree(dst3, n_pad):
    nchunks = dst3.shape[1]
    rpt = n_pad // NS

    mesh = plsc.VectorSubcoreMesh(core_axis_name="c", subcore_axis_name="s")

    @functools.partial(
        pl.kernel,
        out_type=jax.ShapeDtypeStruct((NC, n_pad), jnp.float32),
        mesh=mesh,
        scratch_types=[
            pltpu.VMEM((nchunks, CHUNK), jnp.int32),
            pltpu.VMEM((CHUNK,), jnp.float32),
            pltpu.VMEM((rpt,), jnp.float32),
            pltpu.VMEM_SHARED((n_pad,), jnp.float32),
        ],
    )
    def k(dst_hbm, deg_hbm, dst_v, ones_v, stage_v, deg_sh):
        c = lax.axis_index("c")
        s = lax.axis_index("s")
        slab = c * NS + s
        ones = jnp.ones((16,), jnp.float32)
        zeros = jnp.zeros((16,), jnp.float32)
        for u in range(CHUNK // 16):
            ones_v[pl.ds(u * 16, 16)] = ones

        def zbody(r, _):
            stage_v[pl.ds(r * 16, 16)] = zeros
            return ()
        lax.fori_loop(0, rpt // 16, zbody, ())
        pltpu.sync_copy(stage_v, deg_sh.at[pl.ds(s * rpt, rpt)])
        plsc.subcore_barrier()

        pltpu.sync_copy(dst_hbm.at[slab], dst_v)

        def body(j, _):
            pltpu.sync_copy(ones_v, deg_sh.at[dst_v.at[j]], add=True)
            return ()
        lax.fori_loop(0, nchunks, body, ())
        plsc.subcore_barrier()

        pltpu.sync_copy(deg_sh.at[pl.ds(s * rpt, rpt)], stage_v)
        pltpu.sync_copy(stage_v, deg_hbm.at[c, pl.ds(s * rpt, rpt)])

    return k(dst3)


def _sc_scatter(g, packed2, n_pad):
    per_tile = packed2.shape[1]
    nchunks = per_tile // CHUNK
    npairs = nchunks // 2
    rpt = n_pad // NS

    mesh = plsc.VectorSubcoreMesh(core_axis_name="c", subcore_axis_name="s")

    @functools.partial(
        pl.kernel,
        out_type=jax.ShapeDtypeStruct((NC, n_pad, 128), jnp.float32),
        mesh=mesh,
        scratch_types=[
            pltpu.VMEM((per_tile,), jnp.int32),
            pltpu.VMEM((CHUNK,), jnp.int32),
            pltpu.VMEM((CHUNK,), jnp.int32),
            pltpu.VMEM((1, CHUNK), jnp.int32),
            pltpu.VMEM((1, CHUNK), jnp.int32),
            pltpu.VMEM((CHUNK, 128), jnp.float32),
            pltpu.VMEM((CHUNK, 128), jnp.float32),
            pltpu.VMEM_SHARED((n_pad, 128), jnp.float32),
            pltpu.SemaphoreType.DMA,
            pltpu.SemaphoreType.DMA,
        ],
    )
    def k(g_hbm, pk_hbm, out_hbm, pk_v, sa, sb, da, db, buf_a, buf_b,
          acc_sh, sem_a, sem_b):
        c = lax.axis_index("c")
        s = lax.axis_index("s")
        slab = c * NS + s
        zeros = jnp.zeros((16,), jnp.float32)
        mask = jnp.full((16,), (1 << SHIFT) - 1, jnp.int32)

        def zbody(r, _):
            for u in range(8):
                buf_a[r, pl.ds(u * 16, 16)] = zeros
            return ()
        lax.fori_loop(0, CHUNK, zbody, ())
        for q in range(rpt // CHUNK):
            pltpu.sync_copy(buf_a, acc_sh.at[pl.ds(s * rpt + q * CHUNK, CHUNK)])

        pltpu.sync_copy(pk_hbm.at[slab], pk_v)
        plsc.subcore_barrier()

        def unpack(e, sref, dref):
            base = e * CHUNK
            for u in range(CHUNK // 16):
                w = pk_v[pl.ds(base + u * 16, 16)]
                sref[pl.ds(u * 16, 16)] = w & mask
                dref[0, pl.ds(u * 16, 16)] = lax.shift_right_logical(w, SHIFT)

        unpack(0, sa, da)
        pltpu.async_copy(g_hbm.at[sa], buf_a, sem_a)

        def body(j, _):
            e0 = 2 * j
            unpack(e0 + 1, sb, db)
            pltpu.make_async_copy(g_hbm.at[sa], buf_a, sem_a).wait()
            pltpu.async_copy(g_hbm.at[sb], buf_b, sem_b)
            pltpu.sync_copy(buf_a, acc_sh.at[da.at[0]], add=True)

            @pl.when(j + 1 < npairs)
            def _():
                unpack(e0 + 2, sa, da)

            pltpu.make_async_copy(g_hbm.at[sb], buf_b, sem_b).wait()

            @pl.when(j + 1 < npairs)
            def _():
                pltpu.async_copy(g_hbm.at[sa], buf_a, sem_a)

            pltpu.sync_copy(buf_b, acc_sh.at[db.at[0]], add=True)
            return ()
        lax.fori_loop(0, npairs, body, ())
        plsc.subcore_barrier()

        for q in range(rpt // CHUNK):
            base = s * rpt + q * CHUNK
            pltpu.sync_copy(acc_sh.at[pl.ds(base, CHUNK)], buf_a)
            pltpu.sync_copy(buf_a, out_hbm.at[c, pl.ds(base, CHUNK)])

    return k(g, packed2)


def _tc_first(x, w, d0, d1, n_pad):
    grid = (n_pad // BLK,)

    def body(x_ref, w_ref, d0_ref, d1_ref, g_ref, sl_ref):
        deg = d0_ref[...] + d1_ref[...] + 1.0
        dis = lax.rsqrt(deg)
        inv = 1.0 / deg
        h = jnp.dot(x_ref[...], w_ref[...], preferred_element_type=jnp.float32)
        g_ref[...] = h * dis
        sl_ref[...] = h * inv

    return pl.pallas_call(
        body,
        grid=grid,
        in_specs=[
            pl.BlockSpec((BLK, 128), lambda i: (i, 0)),
            pl.BlockSpec((128, 128), lambda i: (0, 0)),
            pl.BlockSpec((BLK, 1), lambda i: (i, 0)),
            pl.BlockSpec((BLK, 1), lambda i: (i, 0)),
        ],
        out_specs=[
            pl.BlockSpec((BLK, 128), lambda i: (i, 0)),
            pl.BlockSpec((BLK, 128), lambda i: (i, 0)),
        ],
        out_shape=[
            jax.ShapeDtypeStruct((n_pad, 128), jnp.float32),
            jax.ShapeDtypeStruct((n_pad, 128), jnp.float32),
        ],
    )(x, w, d0, d1)


def _tc_mid(sp, sl, b, w, d0, d1, n_pad):
    grid = (n_pad // BLK,)

    def body(sp_ref, sl_ref, b_ref, w_ref, d0_ref, d1_ref, g_ref, sl2_ref):
        deg = d0_ref[...] + d1_ref[...] + 1.0
        dis = lax.rsqrt(deg)
        inv = 1.0 / deg
        o = (sp_ref[0] + sp_ref[1]) * dis + sl_ref[...] + b_ref[...]
        h = jnp.dot(o, w_ref[...], preferred_element_type=jnp.float32)
        g_ref[...] = h * dis
        sl2_ref[...] = h * inv

    return pl.pallas_call(
        body,
        grid=grid,
        in_specs=[
            pl.BlockSpec((2, BLK, 128), lambda i: (0, i, 0)),
            pl.BlockSpec((BLK, 128), lambda i: (i, 0)),
            pl.BlockSpec((1, 128), lambda i: (0, 0)),
            pl.BlockSpec((128, 128), lambda i: (0, 0)),
            pl.BlockSpec((BLK, 1), lambda i: (i, 0)),
            pl.BlockSpec((BLK, 1), lambda i: (i, 0)),
        ],
        out_specs=[
            pl.BlockSpec((BLK, 128), lambda i: (i, 0)),
            pl.BlockSpec((BLK, 128), lambda i: (i, 0)),
        ],
        out_shape=[
            jax.ShapeDtypeStruct((n_pad, 128), jnp.float32),
            jax.ShapeDtypeStruct((n_pad, 128), jnp.float32),
        ],
    )(sp, sl, b, w, d0, d1)


def _tc_last(sp, sl, b, d0, d1, n_pad):
    grid = (n_pad // BLK,)

    def body(sp_ref, sl_ref, b_ref, d0_ref, d1_ref, o_ref):
        deg = d0_ref[...] + d1_ref[...] + 1.0
        dis = lax.rsqrt(deg)
        o_ref[...] = (sp_ref[0] + sp_ref[1]) * dis + sl_ref[...] + b_ref[...]

    return pl.pallas_call(
        body,
        grid=grid,
        in_specs=[
            pl.BlockSpec((2, BLK, 128), lambda i: (0, i, 0)),
            pl.BlockSpec((BLK, 128), lambda i: (i, 0)),
            pl.BlockSpec((1, 128), lambda i: (0, 0)),
            pl.BlockSpec((BLK, 1), lambda i: (i, 0)),
            pl.BlockSpec((BLK, 1), lambda i: (i, 0)),
        ],
        out_specs=pl.BlockSpec((BLK, 128), lambda i: (i, 0)),
        out_shape=jax.ShapeDtypeStruct((n_pad, 128), jnp.float32),
    )(sp, sl, b, d0, d1)


def kernel(x, edge_index, W1, b1, W2, b2):
    n, d = x.shape
    e = edge_index.shape[1]
    n_pad = _round_up(n + 1, BLK)

    src = edge_index[0].astype(jnp.int32)
    dst = edge_index[1].astype(jnp.int32)

    per_tile = _round_up(_round_up(e, NW) // NW, 2 * CHUNK)
    e_pad = per_tile * NW
    src = jnp.pad(src, (0, e_pad - e))
    dst = jnp.pad(dst, (0, e_pad - e), constant_values=n)
    dst3 = dst.reshape(NW, per_tile // CHUNK, CHUNK)
    packed2 = ((dst << SHIFT) | src).reshape(NW, per_tile)

    x_pad = jnp.pad(x, ((0, n_pad - n), (0, 0)))
    b1r = b1.reshape(1, 128)
    b2r = b2.reshape(1, 128)

    deg_p = _sc_degree(dst3, n_pad)
    d0 = deg_p[0].reshape(n_pad, 1)
    d1 = deg_p[1].reshape(n_pad, 1)

    g1, sl1 = _tc_first(x_pad, W1, d0, d1, n_pad)
    sp1 = _sc_scatter(g1, packed2, n_pad)
    g2, sl2 = _tc_mid(sp1, sl1, b1r, W2, d0, d1, n_pad)
    sp2 = _sc_scatter(g2, packed2, n_pad)
    out = _tc_last(sp2, sl2, b2r, d0, d1, n_pad)
    return out[:n]


# speedup vs baseline: 1.0377x; 1.0377x over previous
import functools

import jax
import jax.numpy as jnp
from jax import lax
from jax.experimental import pallas as pl
from jax.experimental.pallas import tpu as pltpu
from jax.experimental.pallas import tpu_sc as plsc

NC = 2
NS = 16
NW = NC * NS
CHUNK = 128
BLK = 1024
SHIFT = 15


def _round_up(a, b):
    return (a + b - 1) // b * b


def _sc_degree(dst3, n_pad):
    nchunks = dst3.shape[1]
    rpt = n_pad // NS

    mesh = plsc.VectorSubcoreMesh(core_axis_name="c", subcore_axis_name="s")

    @functools.partial(
        pl.kernel,
        out_type=jax.ShapeDtypeStruct((NC, n_pad), jnp.float32),
        mesh=mesh,
        scratch_types=[
            pltpu.VMEM((nchunks, CHUNK), jnp.int32),
            pltpu.VMEM((CHUNK,), jnp.float32),
            pltpu.VMEM((rpt,), jnp.float32),
            pltpu.VMEM_SHARED((n_pad,), jnp.float32),
        ],
    )
    def k(dst_hbm, deg_hbm, dst_v, ones_v, stage_v, deg_sh):
        c = lax.axis_index("c")
        s = lax.axis_index("s")
        slab = c * NS + s
        ones = jnp.ones((16,), jnp.float32)
        zeros = jnp.zeros((16,), jnp.float32)
        for u in range(CHUNK // 16):
            ones_v[pl.ds(u * 16, 16)] = ones

        def zbody(r, _):
            stage_v[pl.ds(r * 16, 16)] = zeros
            return ()
        lax.fori_loop(0, rpt // 16, zbody, ())
        pltpu.sync_copy(stage_v, deg_sh.at[pl.ds(s * rpt, rpt)])
        plsc.subcore_barrier()

        pltpu.sync_copy(dst_hbm.at[slab], dst_v)

        def body(j, _):
            pltpu.sync_copy(ones_v, deg_sh.at[dst_v.at[j]], add=True)
            return ()
        lax.fori_loop(0, nchunks, body, ())
        plsc.subcore_barrier()

        pltpu.sync_copy(deg_sh.at[pl.ds(s * rpt, rpt)], stage_v)
        pltpu.sync_copy(stage_v, deg_hbm.at[c, pl.ds(s * rpt, rpt)])

    return k(dst3)


def _sc_scatter(g, packed2, n_pad):
    per_tile = packed2.shape[1]
    nchunks = per_tile // CHUNK
    niter = nchunks // 4
    rpt = n_pad // NS

    mesh = plsc.VectorSubcoreMesh(core_axis_name="c", subcore_axis_name="s")

    @functools.partial(
        pl.kernel,
        out_type=jax.ShapeDtypeStruct((NC, n_pad, 128), jnp.float32),
        mesh=mesh,
        scratch_types=[
            pltpu.VMEM((per_tile,), jnp.int32),
            pltpu.VMEM((CHUNK,), jnp.int32),
            pltpu.VMEM((CHUNK,), jnp.int32),
            pltpu.VMEM((CHUNK,), jnp.int32),
            pltpu.VMEM((CHUNK,), jnp.int32),
            pltpu.VMEM((1, CHUNK), jnp.int32),
            pltpu.VMEM((1, CHUNK), jnp.int32),
            pltpu.VMEM((1, CHUNK), jnp.int32),
            pltpu.VMEM((1, CHUNK), jnp.int32),
            pltpu.VMEM((CHUNK, 128), jnp.float32),
            pltpu.VMEM((CHUNK, 128), jnp.float32),
            pltpu.VMEM_SHARED((n_pad, 128), jnp.float32),
            pltpu.SemaphoreType.DMA,
            pltpu.SemaphoreType.DMA,
            pltpu.SemaphoreType.DMA,
            pltpu.SemaphoreType.DMA,
        ],
    )
    def k(g_hbm, pk_hbm, out_hbm, pk_v, s0, s1, s2, s3, d0, d1, d2, d3,
          buf_a, buf_b, acc_sh, gs0, gs1, as0, as1):
        sidx = [s0, s1, s2, s3]
        didx = [d0, d1, d2, d3]
        bufs = [buf_a, buf_b]
        gsem = [gs0, gs1]
        asem = [as0, as1]
        c = lax.axis_index("c")
        s = lax.axis_index("s")
        slab = c * NS + s
        zeros = jnp.zeros((16,), jnp.float32)
        mask = jnp.full((16,), (1 << SHIFT) - 1, jnp.int32)

        def zbody(r, _):
            for u in range(8):
                buf_a[r, pl.ds(u * 16, 16)] = zeros
            return ()
        lax.fori_loop(0, CHUNK, zbody, ())
        for q in range(rpt // CHUNK):
            pltpu.sync_copy(buf_a, acc_sh.at[pl.ds(s * rpt + q * CHUNK, CHUNK)])

        pltpu.sync_copy(pk_hbm.at[slab], pk_v)
        plsc.subcore_barrier()

        def unpack(e, sref, dref):
            base = e * CHUNK
            for u in range(CHUNK // 16):
                w = pk_v[pl.ds(base + u * 16, 16)]
                sref[pl.ds(u * 16, 16)] = w & mask
                dref[0, pl.ds(u * 16, 16)] = lax.shift_right_logical(w, SHIFT)

        def step(j, r):
            first = r if r < 2 else None

            def stage_a():
                pltpu.make_async_copy(
                    bufs[r % 2], acc_sh.at[didx[(r + 2) % 4].at[0]],
                    asem[r % 2]).wait()

            if first is not None:
                @pl.when(j > 0)
                def _():
                    stage_a()
            else:
                stage_a()

            unpack(4 * j + r, sidx[r], didx[r])
            pltpu.async_copy(g_hbm.at[sidx[r]], bufs[r % 2], gsem[r % 2])

            def stage_d():
                pltpu.make_async_copy(
                    g_hbm.at[sidx[(r + 3) % 4]], bufs[(r + 1) % 2],
                    gsem[(r + 1) % 2]).wait()
                pltpu.async_copy(
                    bufs[(r + 1) % 2], acc_sh.at[didx[(r + 3) % 4].at[0]],
                    asem[(r + 1) % 2], add=True)

            if r == 0:
                @pl.when(j > 0)
                def _():
                    stage_d()
            else:
                stage_d()

        def body(j, _):
            for r in range(4):
                step(j, r)
            return ()
        lax.fori_loop(0, niter, body, ())

        pltpu.make_async_copy(g_hbm.at[sidx[3]], bufs[1], gsem[1]).wait()
        pltpu.async_copy(bufs[1], acc_sh.at[didx[3].at[0]], asem[1], add=True)
        pltpu.make_async_copy(bufs[0], acc_sh.at[didx[2].at[0]], asem[0]).wait()
        pltpu.make_async_copy(bufs[1], acc_sh.at[didx[3].at[0]], asem[1]).wait()
        plsc.subcore_barrier()

        pltpu.sync_copy(acc_sh.at[pl.ds(s * rpt, rpt)],
                        out_hbm.at[c, pl.ds(s * rpt, rpt)])

    return k(g, packed2)


def _tc_first(x, w, d0, d1, n_pad):
    grid = (n_pad // BLK,)

    def body(x_ref, w_ref, d0_ref, d1_ref, g_ref, sl_ref):
        deg = d0_ref[...] + d1_ref[...] + 1.0
        dis = lax.rsqrt(deg)
        inv = 1.0 / deg
        h = jnp.dot(x_ref[...], w_ref[...], preferred_element_type=jnp.float32)
        g_ref[...] = h * dis
        sl_ref[...] = h * inv

    return pl.pallas_call(
        body,
        grid=grid,
        in_specs=[
            pl.BlockSpec((BLK, 128), lambda i: (i, 0)),
            pl.BlockSpec((128, 128), lambda i: (0, 0)),
            pl.BlockSpec((BLK, 1), lambda i: (i, 0)),
            pl.BlockSpec((BLK, 1), lambda i: (i, 0)),
        ],
        out_specs=[
            pl.BlockSpec((BLK, 128), lambda i: (i, 0)),
            pl.BlockSpec((BLK, 128), lambda i: (i, 0)),
        ],
        out_shape=[
            jax.ShapeDtypeStruct((n_pad, 128), jnp.float32),
            jax.ShapeDtypeStruct((n_pad, 128), jnp.float32),
        ],
    )(x, w, d0, d1)


def _tc_mid(sp, sl, b, w, d0, d1, n_pad):
    grid = (n_pad // BLK,)

    def body(sp_ref, sl_ref, b_ref, w_ref, d0_ref, d1_ref, g_ref, sl2_ref):
        deg = d0_ref[...] + d1_ref[...] + 1.0
        dis = lax.rsqrt(deg)
        inv = 1.0 / deg
        o = (sp_ref[0] + sp_ref[1]) * dis + sl_ref[...] + b_ref[...]
        h = jnp.dot(o, w_ref[...], preferred_element_type=jnp.float32)
        g_ref[...] = h * dis
        sl2_ref[...] = h * inv

    return pl.pallas_call(
        body,
        grid=grid,
        in_specs=[
            pl.BlockSpec((2, BLK, 128), lambda i: (0, i, 0)),
            pl.BlockSpec((BLK, 128), lambda i: (i, 0)),
            pl.BlockSpec((1, 128), lambda i: (0, 0)),
            pl.BlockSpec((128, 128), lambda i: (0, 0)),
            pl.BlockSpec((BLK, 1), lambda i: (i, 0)),
            pl.BlockSpec((BLK, 1), lambda i: (i, 0)),
        ],
        out_specs=[
            pl.BlockSpec((BLK, 128), lambda i: (i, 0)),
            pl.BlockSpec((BLK, 128), lambda i: (i, 0)),
        ],
        out_shape=[
            jax.ShapeDtypeStruct((n_pad, 128), jnp.float32),
            jax.ShapeDtypeStruct((n_pad, 128), jnp.float32),
        ],
    )(sp, sl, b, w, d0, d1)


def _tc_last(sp, sl, b, d0, d1, n_pad):
    grid = (n_pad // BLK,)

    def body(sp_ref, sl_ref, b_ref, d0_ref, d1_ref, o_ref):
        deg = d0_ref[...] + d1_ref[...] + 1.0
        dis = lax.rsqrt(deg)
        o_ref[...] = (sp_ref[0] + sp_ref[1]) * dis + sl_ref[...] + b_ref[...]

    return pl.pallas_call(
        body,
        grid=grid,
        in_specs=[
            pl.BlockSpec((2, BLK, 128), lambda i: (0, i, 0)),
            pl.BlockSpec((BLK, 128), lambda i: (i, 0)),
            pl.BlockSpec((1, 128), lambda i: (0, 0)),
            pl.BlockSpec((BLK, 1), lambda i: (i, 0)),
            pl.BlockSpec((BLK, 1), lambda i: (i, 0)),
        ],
        out_specs=pl.BlockSpec((BLK, 128), lambda i: (i, 0)),
        out_shape=jax.ShapeDtypeStruct((n_pad, 128), jnp.float32),
    )(sp, sl, b, d0, d1)


def kernel(x, edge_index, W1, b1, W2, b2):
    n, d = x.shape
    e = edge_index.shape[1]
    n_pad = _round_up(n + 1, BLK)

    src = edge_index[0].astype(jnp.int32)
    dst = edge_index[1].astype(jnp.int32)

    per_tile = _round_up(_round_up(e, NW) // NW, 2 * CHUNK)
    e_pad = per_tile * NW
    src = jnp.pad(src, (0, e_pad - e))
    dst = jnp.pad(dst, (0, e_pad - e), constant_values=n)
    dst3 = dst.reshape(NW, per_tile // CHUNK, CHUNK)
    packed2 = ((dst << SHIFT) | src).reshape(NW, per_tile)

    x_pad = jnp.pad(x, ((0, n_pad - n), (0, 0)))
    b1r = b1.reshape(1, 128)
    b2r = b2.reshape(1, 128)

    deg_p = _sc_degree(dst3, n_pad)
    d0 = deg_p[0].reshape(n_pad, 1)
    d1 = deg_p[1].reshape(n_pad, 1)

    g1, sl1 = _tc_first(x_pad, W1, d0, d1, n_pad)
    sp1 = _sc_scatter(g1, packed2, n_pad)
    g2, sl2 = _tc_mid(sp1, sl1, b1r, W2, d0, d1, n_pad)
    sp2 = _sc_scatter(g2, packed2, n_pad)
    out = _tc_last(sp2, sl2, b2r, d0, d1, n_pad)
    return out[:n]


# speedup vs baseline: 1.1938x; 1.1504x over previous
import functools

import jax
import jax.numpy as jnp
from jax import lax
from jax.experimental import pallas as pl
from jax.experimental.pallas import tpu as pltpu
from jax.experimental.pallas import tpu_sc as plsc

NC = 2
NS = 16
NW = NC * NS
CHUNK = 128
BLK = 1024
SHIFT = 15


def _round_up(a, b):
    return (a + b - 1) // b * b


def _sc_degree(dst3, n_pad):
    nchunks = dst3.shape[1]
    rpt = n_pad // NS

    mesh = plsc.VectorSubcoreMesh(core_axis_name="c", subcore_axis_name="s")

    @functools.partial(
        pl.kernel,
        out_type=jax.ShapeDtypeStruct((NC, n_pad), jnp.float32),
        mesh=mesh,
        scratch_types=[
            pltpu.VMEM((nchunks, CHUNK), jnp.int32),
            pltpu.VMEM((CHUNK,), jnp.float32),
            pltpu.VMEM((rpt,), jnp.float32),
            pltpu.VMEM_SHARED((n_pad,), jnp.float32),
        ],
    )
    def k(dst_hbm, deg_hbm, dst_v, ones_v, stage_v, deg_sh):
        c = lax.axis_index("c")
        s = lax.axis_index("s")
        slab = c * NS + s
        ones = jnp.ones((16,), jnp.float32)
        zeros = jnp.zeros((16,), jnp.float32)
        for u in range(CHUNK // 16):
            ones_v[pl.ds(u * 16, 16)] = ones

        def zbody(r, _):
            stage_v[pl.ds(r * 16, 16)] = zeros
            return ()
        lax.fori_loop(0, rpt // 16, zbody, ())
        pltpu.sync_copy(stage_v, deg_sh.at[pl.ds(s * rpt, rpt)])
        plsc.subcore_barrier()

        pltpu.sync_copy(dst_hbm.at[slab], dst_v)

        def body(j, _):
            pltpu.sync_copy(ones_v, deg_sh.at[dst_v.at[j]], add=True)
            return ()
        lax.fori_loop(0, nchunks, body, ())
        plsc.subcore_barrier()

        pltpu.sync_copy(deg_sh.at[pl.ds(s * rpt, rpt)], stage_v)
        pltpu.sync_copy(stage_v, deg_hbm.at[c, pl.ds(s * rpt, rpt)])

    return k(dst3)


def _sc_scatter(g, packed2, n_pad):
    per_tile = packed2.shape[1]
    nchunks = per_tile // CHUNK
    niter = nchunks // 4
    rpt = n_pad // NS

    mesh = plsc.VectorSubcoreMesh(core_axis_name="c", subcore_axis_name="s")

    @functools.partial(
        pl.kernel,
        out_type=jax.ShapeDtypeStruct((NC, n_pad, 128), jnp.float32),
        mesh=mesh,
        scratch_types=[
            pltpu.VMEM((per_tile,), jnp.int32),
            pltpu.VMEM((CHUNK,), jnp.int32),
            pltpu.VMEM((CHUNK,), jnp.int32),
            pltpu.VMEM((CHUNK,), jnp.int32),
            pltpu.VMEM((CHUNK,), jnp.int32),
            pltpu.VMEM((1, CHUNK), jnp.int32),
            pltpu.VMEM((1, CHUNK), jnp.int32),
            pltpu.VMEM((1, CHUNK), jnp.int32),
            pltpu.VMEM((1, CHUNK), jnp.int32),
            pltpu.VMEM((CHUNK, 128), jnp.float32),
            pltpu.VMEM((CHUNK, 128), jnp.float32),
            pltpu.VMEM_SHARED((n_pad, 128), jnp.float32),
            pltpu.SemaphoreType.DMA,
            pltpu.SemaphoreType.DMA,
            pltpu.SemaphoreType.DMA,
            pltpu.SemaphoreType.DMA,
        ],
    )
    def k(g_hbm, pk_hbm, out_hbm, pk_v, s0, s1, s2, s3, d0, d1, d2, d3,
          buf_a, buf_b, acc_sh, gs0, gs1, as0, as1):
        sidx = [s0, s1, s2, s3]
        didx = [d0, d1, d2, d3]
        bufs = [buf_a, buf_b]
        gsem = [gs0, gs1]
        asem = [as0, as1]
        c = lax.axis_index("c")
        s = lax.axis_index("s")
        slab = c * NS + s
        zeros = jnp.zeros((16,), jnp.float32)
        mask = jnp.full((16,), (1 << SHIFT) - 1, jnp.int32)

        def zbody(r, _):
            for u in range(8):
                buf_a[r, pl.ds(u * 16, 16)] = zeros
            return ()
        lax.fori_loop(0, CHUNK, zbody, ())
        for q in range(rpt // CHUNK):
            pltpu.sync_copy(buf_a, acc_sh.at[pl.ds(s * rpt + q * CHUNK, CHUNK)])

        pltpu.sync_copy(pk_hbm.at[slab], pk_v)
        plsc.subcore_barrier()

        def unpack(e, sref, dref):
            base = e * CHUNK
            for u in range(CHUNK // 16):
                w = pk_v[pl.ds(base + u * 16, 16)]
                sref[pl.ds(u * 16, 16)] = w & mask
                dref[0, pl.ds(u * 16, 16)] = lax.shift_right_logical(w, SHIFT)

        def step(j, r):
            first = r if r < 2 else None

            def stage_a():
                pltpu.make_async_copy(
                    bufs[r % 2], acc_sh.at[didx[(r + 2) % 4].at[0]],
                    asem[r % 2]).wait()

            if first is not None:
                @pl.when(j > 0)
                def _():
                    stage_a()
            else:
                stage_a()

            unpack(4 * j + r, sidx[r], didx[r])
            pltpu.async_copy(g_hbm.at[sidx[r]], bufs[r % 2], gsem[r % 2])

            def stage_d():
                pltpu.make_async_copy(
                    g_hbm.at[sidx[(r + 3) % 4]], bufs[(r + 1) % 2],
                    gsem[(r + 1) % 2]).wait()
                pltpu.async_copy(
                    bufs[(r + 1) % 2], acc_sh.at[didx[(r + 3) % 4].at[0]],
                    asem[(r + 1) % 2], add=True)

            if r == 0:
                @pl.when(j > 0)
                def _():
                    stage_d()
            else:
                stage_d()

        def body(j, _):
            for r in range(4):
                step(j, r)
            return ()
        lax.fori_loop(0, niter, body, ())

        pltpu.make_async_copy(g_hbm.at[sidx[3]], bufs[1], gsem[1]).wait()
        pltpu.async_copy(bufs[1], acc_sh.at[didx[3].at[0]], asem[1], add=True)
        pltpu.make_async_copy(bufs[0], acc_sh.at[didx[2].at[0]], asem[0]).wait()
        pltpu.make_async_copy(bufs[1], acc_sh.at[didx[3].at[0]], asem[1]).wait()
        plsc.subcore_barrier()

        pltpu.sync_copy(acc_sh.at[pl.ds(s * rpt, rpt)],
                        out_hbm.at[c, pl.ds(s * rpt, rpt)])

    return k(g, packed2)


def _tc_first(x, w, d0, d1, n_pad):
    grid = (n_pad // BLK,)

    def body(x_ref, w_ref, d0_ref, d1_ref, g_ref, sl_ref):
        deg = d0_ref[...] + d1_ref[...] + 1.0
        dis = lax.rsqrt(deg)
        inv = 1.0 / deg
        h = jnp.dot(x_ref[...], w_ref[...], preferred_element_type=jnp.float32)
        g_ref[...] = h * dis
        sl_ref[...] = h * inv

    return pl.pallas_call(
        body,
        grid=grid,
        in_specs=[
            pl.BlockSpec((BLK, 128), lambda i: (i, 0)),
            pl.BlockSpec((128, 128), lambda i: (0, 0)),
            pl.BlockSpec((BLK, 1), lambda i: (i, 0)),
            pl.BlockSpec((BLK, 1), lambda i: (i, 0)),
        ],
        out_specs=[
            pl.BlockSpec((BLK, 128), lambda i: (i, 0)),
            pl.BlockSpec((BLK, 128), lambda i: (i, 0)),
        ],
        out_shape=[
            jax.ShapeDtypeStruct((n_pad, 128), jnp.float32),
            jax.ShapeDtypeStruct((n_pad, 128), jnp.float32),
        ],
    )(x, w, d0, d1)


def _tc_mid(sp, sl, b, w, d0, d1, n_pad):
    grid = (n_pad // BLK,)

    def body(sp_ref, sl_ref, b_ref, w_ref, d0_ref, d1_ref, g_ref, sl2_ref):
        deg = d0_ref[...] + d1_ref[...] + 1.0
        dis = lax.rsqrt(deg)
        inv = 1.0 / deg
        o = (sp_ref[0] + sp_ref[1]) * dis + sl_ref[...] + b_ref[...]
        h = jnp.dot(o, w_ref[...], preferred_element_type=jnp.float32)
        g_ref[...] = h * dis
        sl2_ref[...] = h * inv

    return pl.pallas_call(
        body,
        grid=grid,
        in_specs=[
            pl.BlockSpec((2, BLK, 128), lambda i: (0, i, 0)),
            pl.BlockSpec((BLK, 128), lambda i: (i, 0)),
            pl.BlockSpec((1, 128), lambda i: (0, 0)),
            pl.BlockSpec((128, 128), lambda i: (0, 0)),
            pl.BlockSpec((BLK, 1), lambda i: (i, 0)),
            pl.BlockSpec((BLK, 1), lambda i: (i, 0)),
        ],
        out_specs=[
            pl.BlockSpec((BLK, 128), lambda i: (i, 0)),
            pl.BlockSpec((BLK, 128), lambda i: (i, 0)),
        ],
        out_shape=[
            jax.ShapeDtypeStruct((n_pad, 128), jnp.float32),
            jax.ShapeDtypeStruct((n_pad, 128), jnp.float32),
        ],
    )(sp, sl, b, w, d0, d1)


def _tc_last(sp, sl, b, d0, d1, n_pad):
    grid = (n_pad // BLK,)

    def body(sp_ref, sl_ref, b_ref, d0_ref, d1_ref, o_ref):
        deg = d0_ref[...] + d1_ref[...] + 1.0
        dis = lax.rsqrt(deg)
        o_ref[...] = (sp_ref[0] + sp_ref[1]) * dis + sl_ref[...] + b_ref[...]

    return pl.pallas_call(
        body,
        grid=grid,
        in_specs=[
            pl.BlockSpec((2, BLK, 128), lambda i: (0, i, 0)),
            pl.BlockSpec((BLK, 128), lambda i: (i, 0)),
            pl.BlockSpec((1, 128), lambda i: (0, 0)),
            pl.BlockSpec((BLK, 1), lambda i: (i, 0)),
            pl.BlockSpec((BLK, 1), lambda i: (i, 0)),
        ],
        out_specs=pl.BlockSpec((BLK, 128), lambda i: (i, 0)),
        out_shape=jax.ShapeDtypeStruct((n_pad, 128), jnp.float32),
    )(sp, sl, b, d0, d1)


def kernel(x, edge_index, W1, b1, W2, b2):
    n, d = x.shape
    e = edge_index.shape[1]
    n_pad = _round_up(n + 1, BLK)

    src = edge_index[0].astype(jnp.int32)
    dst = edge_index[1].astype(jnp.int32)

    spare = n_pad - n
    e1 = _round_up(e, NW)
    pad_flat = n + (jnp.arange(e1 - e, dtype=jnp.int32) % spare)
    src1 = jnp.concatenate([src, jnp.zeros((e1 - e,), jnp.int32)])
    dst1 = jnp.concatenate([dst, pad_flat])
    per_real = e1 // NW
    per_tile = _round_up(per_real, 2 * CHUNK)
    extra = per_tile - per_real
    pad_dst = n + (jnp.arange(extra, dtype=jnp.int32) % spare)
    src2 = jnp.pad(src1.reshape(NW, per_real), ((0, 0), (0, extra)))
    dst2 = jnp.concatenate(
        [dst1.reshape(NW, per_real),
         jnp.broadcast_to(pad_dst, (NW, extra))], axis=1)
    dst3 = dst2.reshape(NW, per_tile // CHUNK, CHUNK)
    packed2 = (dst2 << SHIFT) | src2

    x_pad = jnp.pad(x, ((0, n_pad - n), (0, 0)))
    b1r = b1.reshape(1, 128)
    b2r = b2.reshape(1, 128)

    deg_p = _sc_degree(dst3, n_pad)
    d0 = deg_p[0].reshape(n_pad, 1)
    d1 = deg_p[1].reshape(n_pad, 1)

    g1, sl1 = _tc_first(x_pad, W1, d0, d1, n_pad)
    sp1 = _sc_scatter(g1, packed2, n_pad)
    g2, sl2 = _tc_mid(sp1, sl1, b1r, W2, d0, d1, n_pad)
    sp2 = _sc_scatter(g2, packed2, n_pad)
    out = _tc_last(sp2, sl2, b2r, d0, d1, n_pad)
    return out[:n]


# speedup vs baseline: 3.3915x; 2.8409x over previous
import functools

import jax
import jax.numpy as jnp
from jax import lax
from jax.experimental import pallas as pl
from jax.experimental.pallas import tpu as pltpu
from jax.experimental.pallas import tpu_sc as plsc

NC = 2
NS = 16
NW = NC * NS
CHUNK = 128
BLK = 1024
SHIFT = 15


def _round_up(a, b):
    return (a + b - 1) // b * b


def _sc_degree(dst3, n_pad):
    nchunks = dst3.shape[1]
    rpt = n_pad // NS

    mesh = plsc.VectorSubcoreMesh(core_axis_name="c", subcore_axis_name="s")

    @functools.partial(
        pl.kernel,
        out_type=jax.ShapeDtypeStruct((NC, n_pad), jnp.float32),
        mesh=mesh,
        scratch_types=[
            pltpu.VMEM((nchunks, CHUNK), jnp.int32),
            pltpu.VMEM((CHUNK,), jnp.float32),
            pltpu.VMEM((rpt,), jnp.float32),
            pltpu.VMEM_SHARED((n_pad,), jnp.float32),
        ],
    )
    def k(dst_hbm, deg_hbm, dst_v, ones_v, stage_v, deg_sh):
        c = lax.axis_index("c")
        s = lax.axis_index("s")
        slab = c * NS + s
        ones = jnp.ones((16,), jnp.float32)
        zeros = jnp.zeros((16,), jnp.float32)
        for u in range(CHUNK // 16):
            ones_v[pl.ds(u * 16, 16)] = ones

        def zbody(r, _):
            stage_v[pl.ds(r * 16, 16)] = zeros
            return ()
        lax.fori_loop(0, rpt // 16, zbody, ())
        pltpu.sync_copy(stage_v, deg_sh.at[pl.ds(s * rpt, rpt)])
        plsc.subcore_barrier()

        pltpu.sync_copy(dst_hbm.at[slab], dst_v)

        def body(j, _):
            pltpu.sync_copy(ones_v, deg_sh.at[dst_v.at[j]], add=True)
            return ()
        lax.fori_loop(0, nchunks, body, ())
        plsc.subcore_barrier()

        pltpu.sync_copy(deg_sh.at[pl.ds(s * rpt, rpt)], stage_v)
        pltpu.sync_copy(stage_v, deg_hbm.at[c, pl.ds(s * rpt, rpt)])

    return k(dst3)


def _sc_scatter(g, packed2, n_pad):
    per_tile = packed2.shape[1]
    nchunks = per_tile // CHUNK
    niter = nchunks // 4
    rpt = n_pad // NS

    mesh = plsc.VectorSubcoreMesh(core_axis_name="c", subcore_axis_name="s")

    @functools.partial(
        pl.kernel,
        out_type=jax.ShapeDtypeStruct((NC, n_pad, 128), jnp.float32),
        mesh=mesh,
        scratch_types=[
            pltpu.VMEM((per_tile,), jnp.int32),
            pltpu.VMEM((CHUNK,), jnp.int32),
            pltpu.VMEM((CHUNK,), jnp.int32),
            pltpu.VMEM((CHUNK,), jnp.int32),
            pltpu.VMEM((CHUNK,), jnp.int32),
            pltpu.VMEM((1, CHUNK), jnp.int32),
            pltpu.VMEM((1, CHUNK), jnp.int32),
            pltpu.VMEM((1, CHUNK), jnp.int32),
            pltpu.VMEM((1, CHUNK), jnp.int32),
            pltpu.VMEM((CHUNK, 128), jnp.float32),
            pltpu.VMEM((CHUNK, 128), jnp.float32),
            pltpu.VMEM_SHARED((n_pad, 128), jnp.float32),
            pltpu.SemaphoreType.DMA,
            pltpu.SemaphoreType.DMA,
            pltpu.SemaphoreType.DMA,
            pltpu.SemaphoreType.DMA,
        ],
    )
    def k(g_hbm, pk_hbm, out_hbm, pk_v, s0, s1, s2, s3, d0, d1, d2, d3,
          buf_a, buf_b, acc_sh, gs0, gs1, as0, as1):
        sidx = [s0, s1, s2, s3]
        didx = [d0, d1, d2, d3]
        bufs = [buf_a, buf_b]
        gsem = [gs0, gs1]
        asem = [as0, as1]
        c = lax.axis_index("c")
        s = lax.axis_index("s")
        slab = c * NS + s
        zeros = jnp.zeros((16,), jnp.float32)
        mask = jnp.full((16,), (1 << SHIFT) - 1, jnp.int32)

        def zbody(r, _):
            for u in range(8):
                buf_a[r, pl.ds(u * 16, 16)] = zeros
            return ()
        lax.fori_loop(0, CHUNK, zbody, ())
        for q in range(rpt // CHUNK):
            pltpu.sync_copy(buf_a, acc_sh.at[pl.ds(s * rpt + q * CHUNK, CHUNK)])

        pltpu.sync_copy(pk_hbm.at[slab], pk_v)
        plsc.subcore_barrier()

        def unpack(e, sref, dref):
            base = e * CHUNK
            for u in range(CHUNK // 16):
                w = pk_v[pl.ds(base + u * 16, 16)]
                sref[pl.ds(u * 16, 16)] = w & mask
                dref[0, pl.ds(u * 16, 16)] = lax.shift_right_logical(w, SHIFT)

        def step(j, r):
            first = r if r < 2 else None

            def stage_a():
                pltpu.make_async_copy(
                    bufs[r % 2], acc_sh.at[didx[(r + 2) % 4].at[0]],
                    asem[r % 2]).wait()

            if first is not None:
                @pl.when(j > 0)
                def _():
                    stage_a()
            else:
                stage_a()

            unpack(4 * j + r, sidx[r], didx[r])
            pltpu.async_copy(g_hbm.at[sidx[r]], bufs[r % 2], gsem[r % 2])

            def stage_d():
                pltpu.make_async_copy(
                    g_hbm.at[sidx[(r + 3) % 4]], bufs[(r + 1) % 2],
                    gsem[(r + 1) % 2]).wait()
                pltpu.async_copy(
                    bufs[(r + 1) % 2], acc_sh.at[didx[(r + 3) % 4].at[0]],
                    asem[(r + 1) % 2], add=True)

            if r == 0:
                @pl.when(j > 0)
                def _():
                    stage_d()
            else:
                stage_d()

        def body(j, _):
            for r in range(4):
                step(j, r)
            return ()
        lax.fori_loop(0, niter, body, ())

        pltpu.make_async_copy(g_hbm.at[sidx[3]], bufs[1], gsem[1]).wait()
        pltpu.async_copy(bufs[1], acc_sh.at[didx[3].at[0]], asem[1], add=True)
        pltpu.make_async_copy(bufs[0], acc_sh.at[didx[2].at[0]], asem[0]).wait()
        pltpu.make_async_copy(bufs[1], acc_sh.at[didx[3].at[0]], asem[1]).wait()
        plsc.subcore_barrier()

        pltpu.sync_copy(acc_sh.at[pl.ds(s * rpt, rpt)],
                        out_hbm.at[c, pl.ds(s * rpt, rpt)])

    return k(g, packed2)


def _tc_first(x, w, d0, d1, n_pad):
    grid = (n_pad // BLK,)

    def body(x_ref, w_ref, d0_ref, d1_ref, g_ref, sl_ref):
        deg = d0_ref[...] + d1_ref[...] + 1.0
        dis = lax.rsqrt(deg)
        inv = 1.0 / deg
        h = jnp.dot(x_ref[...], w_ref[...], preferred_element_type=jnp.float32)
        g_ref[...] = h * dis
        sl_ref[...] = h * inv

    return pl.pallas_call(
        body,
        grid=grid,
        in_specs=[
            pl.BlockSpec((BLK, 128), lambda i: (i, 0)),
            pl.BlockSpec((128, 128), lambda i: (0, 0)),
            pl.BlockSpec((BLK, 1), lambda i: (i, 0)),
            pl.BlockSpec((BLK, 1), lambda i: (i, 0)),
        ],
        out_specs=[
            pl.BlockSpec((BLK, 128), lambda i: (i, 0)),
            pl.BlockSpec((BLK, 128), lambda i: (i, 0)),
        ],
        out_shape=[
            jax.ShapeDtypeStruct((n_pad, 128), jnp.float32),
            jax.ShapeDtypeStruct((n_pad, 128), jnp.float32),
        ],
    )(x, w, d0, d1)


def _tc_mid(sp, sl, b, w, d0, d1, n_pad):
    grid = (n_pad // BLK,)

    def body(sp_ref, sl_ref, b_ref, w_ref, d0_ref, d1_ref, g_ref, sl2_ref):
        deg = d0_ref[...] + d1_ref[...] + 1.0
        dis = lax.rsqrt(deg)
        inv = 1.0 / deg
        o = (sp_ref[0] + sp_ref[1]) * dis + sl_ref[...] + b_ref[...]
        h = jnp.dot(o, w_ref[...], preferred_element_type=jnp.float32)
        g_ref[...] = h * dis
        sl2_ref[...] = h * inv

    return pl.pallas_call(
        body,
        grid=grid,
        in_specs=[
            pl.BlockSpec((2, BLK, 128), lambda i: (0, i, 0)),
            pl.BlockSpec((BLK, 128), lambda i: (i, 0)),
            pl.BlockSpec((1, 128), lambda i: (0, 0)),
            pl.BlockSpec((128, 128), lambda i: (0, 0)),
            pl.BlockSpec((BLK, 1), lambda i: (i, 0)),
            pl.BlockSpec((BLK, 1), lambda i: (i, 0)),
        ],
        out_specs=[
            pl.BlockSpec((BLK, 128), lambda i: (i, 0)),
            pl.BlockSpec((BLK, 128), lambda i: (i, 0)),
        ],
        out_shape=[
            jax.ShapeDtypeStruct((n_pad, 128), jnp.float32),
            jax.ShapeDtypeStruct((n_pad, 128), jnp.float32),
        ],
    )(sp, sl, b, w, d0, d1)


def _tc_last(sp, sl, b, d0, d1, n_pad):
    grid = (n_pad // BLK,)

    def body(sp_ref, sl_ref, b_ref, d0_ref, d1_ref, o_ref):
        deg = d0_ref[...] + d1_ref[...] + 1.0
        dis = lax.rsqrt(deg)
        o_ref[...] = (sp_ref[0] + sp_ref[1]) * dis + sl_ref[...] + b_ref[...]

    return pl.pallas_call(
        body,
        grid=grid,
        in_specs=[
            pl.BlockSpec((2, BLK, 128), lambda i: (0, i, 0)),
            pl.BlockSpec((BLK, 128), lambda i: (i, 0)),
            pl.BlockSpec((1, 128), lambda i: (0, 0)),
            pl.BlockSpec((BLK, 1), lambda i: (i, 0)),
            pl.BlockSpec((BLK, 1), lambda i: (i, 0)),
        ],
        out_specs=pl.BlockSpec((BLK, 128), lambda i: (i, 0)),
        out_shape=jax.ShapeDtypeStruct((n_pad, 128), jnp.float32),
    )(sp, sl, b, d0, d1)


def kernel(x, edge_index, W1, b1, W2, b2):
    n, d = x.shape
    e = edge_index.shape[1]
    n_pad = _round_up(n + 1, BLK)

    src = edge_index[0].astype(jnp.int32)
    dst = edge_index[1].astype(jnp.int32)

    spare = n_pad - n
    e1 = _round_up(e, NW)
    pad_flat = n + (jnp.arange(e1 - e, dtype=jnp.int32) % spare)
    src1 = jnp.concatenate([src, jnp.zeros((e1 - e,), jnp.int32)])
    dst1 = jnp.concatenate([dst, pad_flat])
    per_real = e1 // NW
    per_tile = _round_up(per_real, 2 * CHUNK)
    extra = per_tile - per_real
    pad_dst = n + (jnp.arange(extra, dtype=jnp.int32) % spare)
    src2 = jnp.concatenate(
        [src1.reshape(NW, per_real),
         jnp.broadcast_to(pad_dst, (NW, extra))], axis=1)
    dst2 = jnp.concatenate(
        [dst1.reshape(NW, per_real),
         jnp.broadcast_to(pad_dst, (NW, extra))], axis=1)
    dst3 = dst2.reshape(NW, per_tile // CHUNK, CHUNK)
    packed2 = (dst2 << SHIFT) | src2

    x_pad = jnp.pad(x, ((0, n_pad - n), (0, 0)))
    b1r = b1.reshape(1, 128)
    b2r = b2.reshape(1, 128)

    deg_p = _sc_degree(dst3, n_pad)
    d0 = deg_p[0].reshape(n_pad, 1)
    d1 = deg_p[1].reshape(n_pad, 1)

    g1, sl1 = _tc_first(x_pad, W1, d0, d1, n_pad)
    sp1 = _sc_scatter(g1, packed2, n_pad)
    g2, sl2 = _tc_mid(sp1, sl1, b1r, W2, d0, d1, n_pad)
    sp2 = _sc_scatter(g2, packed2, n_pad)
    out = _tc_last(sp2, sl2, b2r, d0, d1, n_pad)
    return out[:n]


# speedup vs baseline: 3.4028x; 1.0033x over previous
import functools

import jax
import jax.numpy as jnp
from jax import lax
from jax.experimental import pallas as pl
from jax.experimental.pallas import tpu as pltpu
from jax.experimental.pallas import tpu_sc as plsc

NC = 2
NS = 16
NW = NC * NS
CHUNK = 128
BLK = 1024


def _round_up(a, b):
    return (a + b - 1) // b * b


def _sc_degree(dst3, n_pad):
    nchunks = dst3.shape[1]
    rpt = n_pad // NS

    mesh = plsc.VectorSubcoreMesh(core_axis_name="c", subcore_axis_name="s")

    @functools.partial(
        pl.kernel,
        out_type=jax.ShapeDtypeStruct((NC, n_pad), jnp.float32),
        mesh=mesh,
        scratch_types=[
            pltpu.VMEM((nchunks, CHUNK), jnp.int32),
            pltpu.VMEM((CHUNK,), jnp.float32),
            pltpu.VMEM((rpt,), jnp.float32),
            pltpu.VMEM_SHARED((n_pad,), jnp.float32),
        ],
    )
    def k(dst_hbm, deg_hbm, dst_v, ones_v, stage_v, deg_sh):
        c = lax.axis_index("c")
        s = lax.axis_index("s")
        slab = c * NS + s
        ones = jnp.ones((16,), jnp.float32)
        zeros = jnp.zeros((16,), jnp.float32)
        for u in range(CHUNK // 16):
            ones_v[pl.ds(u * 16, 16)] = ones

        def zbody(r, _):
            stage_v[pl.ds(r * 16, 16)] = zeros
            return ()
        lax.fori_loop(0, rpt // 16, zbody, ())
        pltpu.sync_copy(stage_v, deg_sh.at[pl.ds(s * rpt, rpt)])
        plsc.subcore_barrier()

        pltpu.sync_copy(dst_hbm.at[slab], dst_v)

        def body(j, _):
            pltpu.sync_copy(ones_v, deg_sh.at[dst_v.at[j]], add=True)
            return ()
        lax.fori_loop(0, nchunks, body, ())
        plsc.subcore_barrier()

        pltpu.sync_copy(deg_sh.at[pl.ds(s * rpt, rpt)], stage_v)
        pltpu.sync_copy(stage_v, deg_hbm.at[c, pl.ds(s * rpt, rpt)])

    return k(dst3)


def _sc_scatter(g, src3, dst3, n_pad):
    nchunks = src3.shape[1]
    niter = nchunks // 4
    rpt = n_pad // NS

    mesh = plsc.VectorSubcoreMesh(core_axis_name="c", subcore_axis_name="s")

    @functools.partial(
        pl.kernel,
        out_type=jax.ShapeDtypeStruct((NC, n_pad, 128), jnp.float32),
        mesh=mesh,
        scratch_types=[
            pltpu.VMEM((nchunks, CHUNK), jnp.int32),
            pltpu.VMEM((CHUNK,), jnp.int32),
            pltpu.VMEM((CHUNK,), jnp.int32),
            pltpu.VMEM((CHUNK,), jnp.int32),
            pltpu.VMEM((CHUNK,), jnp.int32),
            pltpu.VMEM((CHUNK, 128), jnp.float32),
            pltpu.VMEM((CHUNK, 128), jnp.float32),
            pltpu.VMEM_SHARED((n_pad, 128), jnp.float32),
            pltpu.SemaphoreType.DMA,
            pltpu.SemaphoreType.DMA,
            pltpu.SemaphoreType.DMA,
            pltpu.SemaphoreType.DMA,
            pltpu.SemaphoreType.DMA,
            pltpu.SemaphoreType.DMA,
            pltpu.SemaphoreType.DMA,
            pltpu.SemaphoreType.DMA,
        ],
    )
    def k(g_hbm, src_hbm, dst_hbm, out_hbm, src_v, d0, d1, d2, d3,
          buf_a, buf_b, acc_sh, gs0, gs1, as0, as1, ds0, ds1, ds2, ds3):
        didx = [d0, d1, d2, d3]
        bufs = [buf_a, buf_b]
        gsem = [gs0, gs1]
        asem = [as0, as1]
        dsem = [ds0, ds1, ds2, ds3]
        c = lax.axis_index("c")
        s = lax.axis_index("s")
        slab = c * NS + s
        zeros = jnp.zeros((16,), jnp.float32)

        def zbody(r, _):
            for u in range(8):
                buf_a[r, pl.ds(u * 16, 16)] = zeros
            return ()
        lax.fori_loop(0, CHUNK, zbody, ())
        for q in range(rpt // CHUNK):
            pltpu.sync_copy(buf_a, acc_sh.at[pl.ds(s * rpt + q * CHUNK, CHUNK)])

        pltpu.async_copy(dst_hbm.at[slab, 0], didx[0], dsem[0])
        pltpu.async_copy(dst_hbm.at[slab, 1], didx[1], dsem[1])
        pltpu.sync_copy(src_hbm.at[slab], src_v)
        plsc.subcore_barrier()

        def step(j, r):
            e = 4 * j + r
            first = r if r < 2 else None

            def stage_a():
                pltpu.make_async_copy(
                    bufs[r % 2], acc_sh.at[didx[(r + 2) % 4]],
                    asem[r % 2]).wait()

            if first is not None:
                @pl.when(j > 0)
                def _():
                    stage_a()
            else:
                stage_a()

            @pl.when(e + 2 < nchunks)
            def _():
                pltpu.async_copy(dst_hbm.at[slab, e + 2], didx[(r + 2) % 4],
                                 dsem[(r + 2) % 4])

            pltpu.async_copy(g_hbm.at[src_v.at[e]], bufs[r % 2], gsem[r % 2])

            def stage_d():
                pltpu.make_async_copy(
                    g_hbm.at[src_v.at[e - 1]], bufs[(r + 1) % 2],
                    gsem[(r + 1) % 2]).wait()
                pltpu.make_async_copy(
                    dst_hbm.at[slab, e - 1], didx[(r + 3) % 4],
                    dsem[(r + 3) % 4]).wait()
                pltpu.async_copy(
                    bufs[(r + 1) % 2], acc_sh.at[didx[(r + 3) % 4]],
                    asem[(r + 1) % 2], add=True)

            if r == 0:
                @pl.when(j > 0)
                def _():
                    stage_d()
            else:
                stage_d()

        def body(j, _):
            for r in range(4):
                step(j, r)
            return ()
        lax.fori_loop(0, niter, body, ())

        last = nchunks - 1
        pltpu.make_async_copy(g_hbm.at[src_v.at[last]], bufs[1], gsem[1]).wait()
        pltpu.make_async_copy(dst_hbm.at[slab, last], didx[3], dsem[3]).wait()
        pltpu.async_copy(bufs[1], acc_sh.at[didx[3]], asem[1], add=True)
        pltpu.make_async_copy(bufs[0], acc_sh.at[didx[2]], asem[0]).wait()
        pltpu.make_async_copy(bufs[1], acc_sh.at[didx[3]], asem[1]).wait()
        plsc.subcore_barrier()

        pltpu.sync_copy(acc_sh.at[pl.ds(s * rpt, rpt)],
                        out_hbm.at[c, pl.ds(s * rpt, rpt)])

    return k(g, src3, dst3)


def _tc_first(x, w, d0, d1, n_pad):
    grid = (n_pad // BLK,)

    def body(x_ref, w_ref, d0_ref, d1_ref, g_ref, sl_ref):
        deg = d0_ref[...] + d1_ref[...] + 1.0
        dis = lax.rsqrt(deg)
        inv = 1.0 / deg
        h = jnp.dot(x_ref[...], w_ref[...], preferred_element_type=jnp.float32)
        g_ref[...] = h * dis
        sl_ref[...] = h * inv

    return pl.pallas_call(
        body,
        grid=grid,
        in_specs=[
            pl.BlockSpec((BLK, 128), lambda i: (i, 0)),
            pl.BlockSpec((128, 128), lambda i: (0, 0)),
            pl.BlockSpec((BLK, 1), lambda i: (i, 0)),
            pl.BlockSpec((BLK, 1), lambda i: (i, 0)),
        ],
        out_specs=[
            pl.BlockSpec((BLK, 128), lambda i: (i, 0)),
            pl.BlockSpec((BLK, 128), lambda i: (i, 0)),
        ],
        out_shape=[
            jax.ShapeDtypeStruct((n_pad, 128), jnp.float32),
            jax.ShapeDtypeStruct((n_pad, 128), jnp.float32),
        ],
    )(x, w, d0, d1)


def _tc_mid(sp, sl, b, w, d0, d1, n_pad):
    grid = (n_pad // BLK,)

    def body(sp_ref, sl_ref, b_ref, w_ref, d0_ref, d1_ref, g_ref, sl2_ref):
        deg = d0_ref[...] + d1_ref[...] + 1.0
        dis = lax.rsqrt(deg)
        inv = 1.0 / deg
        o = (sp_ref[0] + sp_ref[1]) * dis + sl_ref[...] + b_ref[...]
        h = jnp.dot(o, w_ref[...], preferred_element_type=jnp.float32)
        g_ref[...] = h * dis
        sl2_ref[...] = h * inv

    return pl.pallas_call(
        body,
        grid=grid,
        in_specs=[
            pl.BlockSpec((2, BLK, 128), lambda i: (0, i, 0)),
            pl.BlockSpec((BLK, 128), lambda i: (i, 0)),
            pl.BlockSpec((1, 128), lambda i: (0, 0)),
            pl.BlockSpec((128, 128), lambda i: (0, 0)),
            pl.BlockSpec((BLK, 1), lambda i: (i, 0)),
            pl.BlockSpec((BLK, 1), lambda i: (i, 0)),
        ],
        out_specs=[
            pl.BlockSpec((BLK, 128), lambda i: (i, 0)),
            pl.BlockSpec((BLK, 128), lambda i: (i, 0)),
        ],
        out_shape=[
            jax.ShapeDtypeStruct((n_pad, 128), jnp.float32),
            jax.ShapeDtypeStruct((n_pad, 128), jnp.float32),
        ],
    )(sp, sl, b, w, d0, d1)


def _tc_last(sp, sl, b, d0, d1, n_pad):
    grid = (n_pad // BLK,)

    def body(sp_ref, sl_ref, b_ref, d0_ref, d1_ref, o_ref):
        deg = d0_ref[...] + d1_ref[...] + 1.0
        dis = lax.rsqrt(deg)
        o_ref[...] = (sp_ref[0] + sp_ref[1]) * dis + sl_ref[...] + b_ref[...]

    return pl.pallas_call(
        body,
        grid=grid,
        in_specs=[
            pl.BlockSpec((2, BLK, 128), lambda i: (0, i, 0)),
            pl.BlockSpec((BLK, 128), lambda i: (i, 0)),
            pl.BlockSpec((1, 128), lambda i: (0, 0)),
            pl.BlockSpec((BLK, 1), lambda i: (i, 0)),
            pl.BlockSpec((BLK, 1), lambda i: (i, 0)),
        ],
        out_specs=pl.BlockSpec((BLK, 128), lambda i: (i, 0)),
        out_shape=jax.ShapeDtypeStruct((n_pad, 128), jnp.float32),
    )(sp, sl, b, d0, d1)


def kernel(x, edge_index, W1, b1, W2, b2):
    n, d = x.shape
    e = edge_index.shape[1]
    n_pad = _round_up(n + 1, BLK)

    src = edge_index[0].astype(jnp.int32)
    dst = edge_index[1].astype(jnp.int32)

    spare = n_pad - n
    e1 = _round_up(e, NW)
    pad_flat = n + (jnp.arange(e1 - e, dtype=jnp.int32) % spare)
    src1 = jnp.concatenate([src, jnp.zeros((e1 - e,), jnp.int32)])
    dst1 = jnp.concatenate([dst, pad_flat])
    per_real = e1 // NW
    per_tile = _round_up(per_real, 2 * CHUNK)
    extra = per_tile - per_real
    pad_dst = n + (jnp.arange(extra, dtype=jnp.int32) % spare)
    src2 = jnp.concatenate(
        [src1.reshape(NW, per_real),
         jnp.broadcast_to(pad_dst, (NW, extra))], axis=1)
    dst2 = jnp.concatenate(
        [dst1.reshape(NW, per_real),
         jnp.broadcast_to(pad_dst, (NW, extra))], axis=1)
    src3 = src2.reshape(NW, per_tile // CHUNK, CHUNK)
    dst3 = dst2.reshape(NW, per_tile // CHUNK, CHUNK)

    x_pad = jnp.pad(x, ((0, n_pad - n), (0, 0)))
    b1r = b1.reshape(1, 128)
    b2r = b2.reshape(1, 128)

    deg_p = _sc_degree(dst3, n_pad)
    d0 = deg_p[0].reshape(n_pad, 1)
    d1 = deg_p[1].reshape(n_pad, 1)

    g1, sl1 = _tc_first(x_pad, W1, d0, d1, n_pad)
    sp1 = _sc_scatter(g1, src3, dst3, n_pad)
    g2, sl2 = _tc_mid(sp1, sl1, b1r, W2, d0, d1, n_pad)
    sp2 = _sc_scatter(g2, src3, dst3, n_pad)
    out = _tc_last(sp2, sl2, b2r, d0, d1, n_pad)
    return out[:n]


# speedup vs baseline: 3.4607x; 1.0170x over previous
import functools

import jax
import jax.numpy as jnp
from jax import lax
from jax.experimental import pallas as pl
from jax.experimental.pallas import tpu as pltpu
from jax.experimental.pallas import tpu_sc as plsc

NC = 2
NS = 16
NW = NC * NS
CHUNK = 128
BLK = 1024


def _round_up(a, b):
    return (a + b - 1) // b * b


def _sc_degree(dst3, n_pad):
    nchunks = dst3.shape[1]
    rpt = n_pad // NS

    mesh = plsc.VectorSubcoreMesh(core_axis_name="c", subcore_axis_name="s")

    @functools.partial(
        pl.kernel,
        out_type=jax.ShapeDtypeStruct((NC, n_pad), jnp.float32),
        mesh=mesh,
        scratch_types=[
            pltpu.VMEM((nchunks, CHUNK), jnp.int32),
            pltpu.VMEM((CHUNK,), jnp.float32),
            pltpu.VMEM((rpt,), jnp.float32),
            pltpu.VMEM_SHARED((n_pad,), jnp.float32),
        ],
    )
    def k(dst_hbm, deg_hbm, dst_v, ones_v, stage_v, deg_sh):
        c = lax.axis_index("c")
        s = lax.axis_index("s")
        slab = c * NS + s
        ones = jnp.ones((16,), jnp.float32)
        zeros = jnp.zeros((16,), jnp.float32)
        for u in range(CHUNK // 16):
            ones_v[pl.ds(u * 16, 16)] = ones

        def zbody(r, _):
            stage_v[pl.ds(r * 16, 16)] = zeros
            return ()
        lax.fori_loop(0, rpt // 16, zbody, ())
        pltpu.sync_copy(stage_v, deg_sh.at[pl.ds(s * rpt, rpt)])
        plsc.subcore_barrier()

        pltpu.sync_copy(dst_hbm.at[slab], dst_v)

        def body(j, _):
            pltpu.sync_copy(ones_v, deg_sh.at[dst_v.at[j]], add=True)
            return ()
        lax.fori_loop(0, nchunks, body, ())
        plsc.subcore_barrier()

        pltpu.sync_copy(deg_sh.at[pl.ds(s * rpt, rpt)], stage_v)
        pltpu.sync_copy(stage_v, deg_hbm.at[c, pl.ds(s * rpt, rpt)])

    return k(dst3)


def _sc_scatter(g, src3, dst3, n_pad):
    nchunks = src3.shape[1]
    niter = nchunks // 4
    rpt = n_pad // NS

    mesh = plsc.VectorSubcoreMesh(core_axis_name="c", subcore_axis_name="s")

    @functools.partial(
        pl.kernel,
        out_type=jax.ShapeDtypeStruct((NC, n_pad, 128), jnp.float32),
        mesh=mesh,
        scratch_types=[
            pltpu.VMEM((nchunks, CHUNK), jnp.int32),
            pltpu.VMEM((CHUNK,), jnp.int32),
            pltpu.VMEM((CHUNK,), jnp.int32),
            pltpu.VMEM((CHUNK,), jnp.int32),
            pltpu.VMEM((CHUNK,), jnp.int32),
            pltpu.VMEM((CHUNK, 128), jnp.float32),
            pltpu.VMEM((CHUNK, 128), jnp.float32),
            pltpu.VMEM_SHARED((n_pad, 128), jnp.float32),
            pltpu.SemaphoreType.DMA,
            pltpu.SemaphoreType.DMA,
            pltpu.SemaphoreType.DMA,
            pltpu.SemaphoreType.DMA,
            pltpu.SemaphoreType.DMA,
            pltpu.SemaphoreType.DMA,
            pltpu.SemaphoreType.DMA,
            pltpu.SemaphoreType.DMA,
        ],
    )
    def k(g_hbm, src_hbm, dst_hbm, out_hbm, src_v, d0, d1, d2, d3,
          buf_a, buf_b, acc_sh, gs0, gs1, as0, as1, ds0, ds1, ds2, ds3):
        didx = [d0, d1, d2, d3]
        bufs = [buf_a, buf_b]
        gsem = [gs0, gs1]
        asem = [as0, as1]
        dsem = [ds0, ds1, ds2, ds3]
        c = lax.axis_index("c")
        s = lax.axis_index("s")
        slab = c * NS + s
        zeros = jnp.zeros((16,), jnp.float32)

        def zbody(r, _):
            for u in range(8):
                buf_a[r, pl.ds(u * 16, 16)] = zeros
            return ()
        lax.fori_loop(0, CHUNK, zbody, ())
        for q in range(rpt // CHUNK):
            pltpu.sync_copy(buf_a, acc_sh.at[pl.ds(s * rpt + q * CHUNK, CHUNK)])

        pltpu.async_copy(dst_hbm.at[slab, 0], didx[0], dsem[0])
        pltpu.async_copy(dst_hbm.at[slab, 1], didx[1], dsem[1])
        pltpu.sync_copy(src_hbm.at[slab], src_v)
        plsc.subcore_barrier()

        def step(j, r):
            e = 4 * j + r
            first = r if r < 2 else None

            def stage_a():
                pltpu.make_async_copy(
                    bufs[r % 2], acc_sh.at[didx[(r + 2) % 4]],
                    asem[r % 2]).wait()

            if first is not None:
                @pl.when(j > 0)
                def _():
                    stage_a()
            else:
                stage_a()

            @pl.when(e + 2 < nchunks)
            def _():
                pltpu.async_copy(dst_hbm.at[slab, e + 2], didx[(r + 2) % 4],
                                 dsem[(r + 2) % 4])

            pltpu.async_copy(g_hbm.at[src_v.at[e]], bufs[r % 2], gsem[r % 2])

            def stage_d():
                pltpu.make_async_copy(
                    g_hbm.at[src_v.at[e - 1]], bufs[(r + 1) % 2],
                    gsem[(r + 1) % 2]).wait()
                pltpu.make_async_copy(
                    dst_hbm.at[slab, e - 1], didx[(r + 3) % 4],
                    dsem[(r + 3) % 4]).wait()
                pltpu.async_copy(
                    bufs[(r + 1) % 2], acc_sh.at[didx[(r + 3) % 4]],
                    asem[(r + 1) % 2], add=True)

            if r == 0:
                @pl.when(j > 0)
                def _():
                    stage_d()
            else:
                stage_d()

        def body(j, _):
            for r in range(4):
                step(j, r)
            return ()
        lax.fori_loop(0, niter, body, ())

        last = nchunks - 1
        pltpu.make_async_copy(g_hbm.at[src_v.at[last]], bufs[1], gsem[1]).wait()
        pltpu.make_async_copy(dst_hbm.at[slab, last], didx[3], dsem[3]).wait()
        pltpu.async_copy(bufs[1], acc_sh.at[didx[3]], asem[1], add=True)
        pltpu.make_async_copy(bufs[0], acc_sh.at[didx[2]], asem[0]).wait()
        pltpu.make_async_copy(bufs[1], acc_sh.at[didx[3]], asem[1]).wait()
        plsc.subcore_barrier()

        pltpu.sync_copy(acc_sh.at[pl.ds(s * rpt, rpt)],
                        out_hbm.at[c, pl.ds(s * rpt, rpt)])

    return k(g, src3, dst3)


def _tc_first(x, w, d0, d1, n_pad):
    grid = (n_pad // BLK,)

    def body(x_ref, w_ref, d0_ref, d1_ref, g_ref, sl_ref):
        deg = d0_ref[...] + d1_ref[...] + 1.0
        dis = lax.rsqrt(deg)
        inv = 1.0 / deg
        h = jnp.dot(x_ref[...], w_ref[...], preferred_element_type=jnp.float32)
        g_ref[...] = h * dis
        sl_ref[...] = h * inv

    return pl.pallas_call(
        body,
        grid=grid,
        in_specs=[
            pl.BlockSpec((BLK, 128), lambda i: (i, 0)),
            pl.BlockSpec((128, 128), lambda i: (0, 0)),
            pl.BlockSpec((BLK, 1), lambda i: (i, 0)),
            pl.BlockSpec((BLK, 1), lambda i: (i, 0)),
        ],
        out_specs=[
            pl.BlockSpec((BLK, 128), lambda i: (i, 0)),
            pl.BlockSpec((BLK, 128), lambda i: (i, 0)),
        ],
        out_shape=[
            jax.ShapeDtypeStruct((n_pad, 128), jnp.float32),
            jax.ShapeDtypeStruct((n_pad, 128), jnp.float32),
        ],
    )(x, w, d0, d1)


def _tc_mid(sp, sl, b, w, d0, d1, n_pad):
    grid = (n_pad // BLK,)

    def body(sp_ref, sl_ref, b_ref, w_ref, d0_ref, d1_ref, g_ref, sl2_ref):
        deg = d0_ref[...] + d1_ref[...] + 1.0
        dis = lax.rsqrt(deg)
        inv = 1.0 / deg
        o = (sp_ref[0] + sp_ref[1]) * dis + sl_ref[...] + b_ref[...]
        h = jnp.dot(o, w_ref[...], preferred_element_type=jnp.float32)
        g_ref[...] = h * dis
        sl2_ref[...] = h * inv

    return pl.pallas_call(
        body,
        grid=grid,
        in_specs=[
            pl.BlockSpec((2, BLK, 128), lambda i: (0, i, 0)),
            pl.BlockSpec((BLK, 128), lambda i: (i, 0)),
            pl.BlockSpec((1, 128), lambda i: (0, 0)),
            pl.BlockSpec((128, 128), lambda i: (0, 0)),
            pl.BlockSpec((BLK, 1), lambda i: (i, 0)),
            pl.BlockSpec((BLK, 1), lambda i: (i, 0)),
        ],
        out_specs=[
            pl.BlockSpec((BLK, 128), lambda i: (i, 0)),
            pl.BlockSpec((BLK, 128), lambda i: (i, 0)),
        ],
        out_shape=[
            jax.ShapeDtypeStruct((n_pad, 128), jnp.float32),
            jax.ShapeDtypeStruct((n_pad, 128), jnp.float32),
        ],
    )(sp, sl, b, w, d0, d1)


def _tc_last(sp, sl, b, d0, d1, n_pad, n):
    grid = (n_pad // BLK,)

    def body(sp_ref, sl_ref, b_ref, d0_ref, d1_ref, o_ref):
        deg = d0_ref[...] + d1_ref[...] + 1.0
        dis = lax.rsqrt(deg)
        o_ref[...] = (sp_ref[0] + sp_ref[1]) * dis + sl_ref[...] + b_ref[...]

    return pl.pallas_call(
        body,
        grid=grid,
        in_specs=[
            pl.BlockSpec((2, BLK, 128), lambda i: (0, i, 0)),
            pl.BlockSpec((BLK, 128), lambda i: (i, 0)),
            pl.BlockSpec((1, 128), lambda i: (0, 0)),
            pl.BlockSpec((BLK, 1), lambda i: (i, 0)),
            pl.BlockSpec((BLK, 1), lambda i: (i, 0)),
        ],
        out_specs=pl.BlockSpec((BLK, 128), lambda i: (i, 0)),
        out_shape=jax.ShapeDtypeStruct((n, 128), jnp.float32),
    )(sp, sl, b, d0, d1)


def kernel(x, edge_index, W1, b1, W2, b2):
    n, d = x.shape
    e = edge_index.shape[1]
    n_pad = _round_up(n + 1, BLK)

    src = edge_index[0].astype(jnp.int32)
    dst = edge_index[1].astype(jnp.int32)

    spare = n_pad - n
    e1 = _round_up(e, NW)
    pad_flat = n + (jnp.arange(e1 - e, dtype=jnp.int32) % spare)
    src1 = jnp.concatenate([src, jnp.zeros((e1 - e,), jnp.int32)])
    dst1 = jnp.concatenate([dst, pad_flat])
    per_real = e1 // NW
    per_tile = _round_up(per_real, 2 * CHUNK)
    extra = per_tile - per_real
    pad_dst = n + (jnp.arange(extra, dtype=jnp.int32) % spare)
    src2 = jnp.concatenate(
        [src1.reshape(NW, per_real),
         jnp.broadcast_to(pad_dst, (NW, extra))], axis=1)
    dst2 = jnp.concatenate(
        [dst1.reshape(NW, per_real),
         jnp.broadcast_to(pad_dst, (NW, extra))], axis=1)
    src3 = src2.reshape(NW, per_tile // CHUNK, CHUNK)
    dst3 = dst2.reshape(NW, per_tile // CHUNK, CHUNK)

    x_pad = jnp.pad(x, ((0, n_pad - n), (0, 0)))
    b1r = b1.reshape(1, 128)
    b2r = b2.reshape(1, 128)

    deg_p = _sc_degree(dst3, n_pad)
    d0 = deg_p[0].reshape(n_pad, 1)
    d1 = deg_p[1].reshape(n_pad, 1)

    g1, sl1 = _tc_first(x_pad, W1, d0, d1, n_pad)
    sp1 = _sc_scatter(g1, src3, dst3, n_pad)
    g2, sl2 = _tc_mid(sp1, sl1, b1r, W2, d0, d1, n_pad)
    sp2 = _sc_scatter(g2, src3, dst3, n_pad)
    return _tc_last(sp2, sl2, b2r, d0, d1, n_pad, n)
